# Initial kernel scaffold; baseline (speedup 1.0000x reference)
import jax
import jax.numpy as jnp
from jax.experimental import pallas as pl


def kernel(pseudo_labels, feat_old_0, feat_0, feat_old_1, feat_1, num_class, num_old_class):
    raise NotImplementedError("write your pallas kernel here")


# speedup vs baseline: 1.7968x; 1.7968x over previous
import functools

import jax
import jax.numpy as jnp
from jax.experimental import pallas as pl
from jax.experimental.pallas import tpu as pltpu

NCLS = 24
LANES = 128


def _scale_body(x_ref, y_ref, lab_ref, sums_ref, cnts_ref, *, s_blk):
    i = pl.program_id(0)

    @pl.when(i == 0)
    def _init():
        sums_ref[...] = jnp.zeros_like(sums_ref)
        cnts_ref[...] = jnp.zeros_like(cnts_ref)

    x = x_ref[0]
    y = y_ref[0]
    mx = jnp.max(x, axis=0, keepdims=True)
    ex = jnp.exp(x - mx)
    sx = jnp.sum(ex, axis=0, keepdims=True)
    my = jnp.max(y, axis=0, keepdims=True)
    ey = jnp.exp(y - my)
    sy = jnp.sum(ey, axis=0, keepdims=True)
    t = jnp.sum(ex * (x - y), axis=0, keepdims=True) / sx
    kl = t - (mx + jnp.log(sx)) + (my + jnp.log(sy))

    lab = lab_ref[0]
    cls = jax.lax.broadcasted_iota(jnp.int32, (NCLS, 1), 0)
    mask = lab == cls
    contrib = jnp.where(mask, kl, jnp.float32(0.0))
    cnt = mask.astype(jnp.float32)
    part_s = jnp.zeros((NCLS, LANES), jnp.float32)
    part_c = jnp.zeros((NCLS, LANES), jnp.float32)
    for j in range(s_blk // LANES):
        part_s = part_s + contrib[:, j * LANES:(j + 1) * LANES]
        part_c = part_c + cnt[:, j * LANES:(j + 1) * LANES]
    sums_ref[...] += part_s
    cnts_ref[...] += part_c


def _scale_call(x, y, lab, s_blk):
    b, c, hw = x.shape
    nb = hw // s_blk
    grid = (b * nb,)
    feat_spec = pl.BlockSpec((1, c, s_blk), lambda i: (i // nb, 0, i % nb))
    lab_spec = pl.BlockSpec((1, 1, s_blk), lambda i: (i, 0, 0))
    acc_spec = pl.BlockSpec((NCLS, LANES), lambda i: (0, 0))
    return pl.pallas_call(
        functools.partial(_scale_body, s_blk=s_blk),
        grid=grid,
        in_specs=[feat_spec, feat_spec, lab_spec],
        out_specs=[acc_spec, acc_spec],
        out_shape=[
            jax.ShapeDtypeStruct((NCLS, LANES), jnp.float32),
            jax.ShapeDtypeStruct((NCLS, LANES), jnp.float32),
        ],
    )(x, y, lab)


def _combine_body(g_ref, s0_ref, c0_ref, s1_ref, c1_ref, out_ref):
    gate = g_ref[:, :1]

    def term(s_ref, c_ref):
        s = jnp.sum(s_ref[...], axis=1, keepdims=True)
        c = jnp.sum(c_ref[...], axis=1, keepdims=True)
        klc = s / jnp.maximum(c, 1.0)
        return jnp.sum(gate * jnp.where(c > 0, klc, jnp.float32(0.0)))

    loss = jnp.float32(1.0) * term(s0_ref, c0_ref) + jnp.float32(2.0) * term(s1_ref, c1_ref)
    out_ref[...] = jnp.full((8, LANES), loss, jnp.float32)


def _combine_call(gate2d, s0, c0, s1, c1):
    spec = pl.BlockSpec((NCLS, LANES), lambda: (0, 0))
    return pl.pallas_call(
        _combine_body,
        in_specs=[spec] * 5,
        out_specs=pl.BlockSpec((8, LANES), lambda: (0, 0)),
        out_shape=jax.ShapeDtypeStruct((8, LANES), jnp.float32),
    )(gate2d, s0, c0, s1, c1)


def kernel(pseudo_labels, feat_old_0, feat_0, feat_old_1, feat_1, num_class, num_old_class):
    b = pseudo_labels.shape[0]

    lab0 = pseudo_labels[:, 0, ::8, ::8].reshape(-1, 1, 512)
    lab1 = pseudo_labels[:, 0, ::16, ::16].reshape(-1, 1, 512)

    x0 = feat_0.reshape(b, 384, 64 * 64)
    y0 = feat_old_0.reshape(b, 384, 64 * 64)
    x1 = feat_1.reshape(b, 768, 32 * 32)
    y1 = feat_old_1.reshape(b, 768, 32 * 32)

    s0, c0 = _scale_call(x0, y0, lab0, 512)
    s1, c1 = _scale_call(x1, y1, lab1, 512)

    cls = jnp.arange(NCLS, dtype=jnp.float32)
    noc = jnp.asarray(num_old_class, jnp.float32)
    nc = jnp.asarray(num_class, jnp.float32)
    gate = jnp.where(
        cls == 0,
        noc / nc,
        jnp.where((cls <= noc) & (cls < 21), jnp.float32(1.0), jnp.float32(0.0)),
    )
    gate2d = jnp.broadcast_to(gate[:, None], (NCLS, LANES))

    out = _combine_call(gate2d, s0, c0, s1, c1)
    return out[0, 0]
